# baseline (speedup 1.0000x reference)
import functools

import jax
import jax.numpy as jnp
from jax import lax
from jax.experimental import pallas as pl
from jax.experimental.pallas import tpu as pltpu
from jax.experimental.pallas import tpu_sc as plsc

D = 64
ADD = 16

NC = 2
NS = 16
NW = NC * NS
CHUNK = 128


def _sc_gather_body(table_hbm, idx_hbm, out_hbm, idx_v, idx2_v, rows_v, gsem,
                    osem):
    l_ctx, b = idx_hbm.shape
    nchunk = l_ctx
    wid = lax.axis_index("s") * NC + lax.axis_index("c")
    col = wid * CHUNK
    pltpu.sync_copy(idx_hbm.at[:, pl.ds(col, CHUNK)], idx_v)
    for j in range(nchunk):
        for g in range(CHUNK // 16):
            v = idx_v[j, pl.ds(g * 16, 16)]
            idx2_v[j, pl.ds(g * 16, 16)] = lax.shift_right_logical(v, 1)
    gh = {}
    oh = {}
    for j in range(nchunk):
        if j >= 4:
            oh[j - 4].wait()
        gh[j] = pltpu.async_copy(table_hbm.at[idx2_v.at[j]], rows_v.at[j % 4],
                                 gsem)
        if j >= 1:
            gh[j - 1].wait()
            oh[j - 1] = pltpu.async_copy(
                rows_v.at[(j - 1) % 4],
                out_hbm.at[pl.ds((j - 1) * b + col, CHUNK)], osem)
    gh[nchunk - 1].wait()
    oh[nchunk - 1] = pltpu.async_copy(
        rows_v.at[(nchunk - 1) % 4],
        out_hbm.at[pl.ds((nchunk - 1) * b + col, CHUNK)], osem)
    for j in range(max(0, nchunk - 4), nchunk):
        oh[j].wait()


def _sc_gather(table2, ids_t):
    l_ctx, b = ids_t.shape
    n_flat = l_ctx * b
    mesh = plsc.VectorSubcoreMesh(core_axis_name="c", subcore_axis_name="s")
    return pl.kernel(
        _sc_gather_body,
        out_type=jax.ShapeDtypeStruct((n_flat, 2 * D), jnp.float32),
        mesh=mesh,
        scratch_types=[
            pltpu.VMEM((l_ctx, CHUNK), jnp.int32),
            pltpu.VMEM((l_ctx, CHUNK), jnp.int32),
            pltpu.VMEM((4, CHUNK, 2 * D), jnp.float32),
            pltpu.SemaphoreType.DMA,
            pltpu.SemaphoreType.DMA,
        ],
    )(table2, ids_t)


def _mm_body(x_ref, par_ref, pad_ref, w_ref, o_ref):
    w1 = w_ref[0:D, :]
    w2 = w_ref[D:, :]
    a = x_ref[:, 0:D]
    bb = x_ref[:, D:]
    out_a = lax.dot_general(
        w1, a, (((0,), (1,)), ((), ())), preferred_element_type=jnp.float32)
    out_b = lax.dot_general(
        w1, bb, (((0,), (1,)), ((), ())), preferred_element_type=jnp.float32)
    p = par_ref[0]
    acc = out_a * (1.0 - p) + out_b * p
    acc += lax.dot_general(
        w2, pad_ref[0], (((0,), (0,)), ((), ())),
        preferred_element_type=jnp.float32)
    o_ref[...] = acc


def _tc_matmul(sym2, par_t, pad_t, w_rev, l_ctx, bn):
    n_flat = sym2.shape[0]
    b = n_flat // l_ctx
    rv = w_rev.shape[1]
    nb = b // bn
    return pl.pallas_call(
        _mm_body,
        grid=(l_ctx, nb),
        in_specs=[
            pl.BlockSpec((bn, 2 * D), lambda l, j: (l * nb + j, 0)),
            pl.BlockSpec((1, 1, bn), lambda l, j: (l, 0, j)),
            pl.BlockSpec((1, ADD, bn), lambda l, j: (l, 0, j)),
            pl.BlockSpec((D + ADD, rv), lambda l, j: (0, 0)),
        ],
        out_specs=pl.BlockSpec((rv, bn), lambda l, j: (l, j)),
        out_shape=jax.ShapeDtypeStruct((l_ctx * rv, b), jnp.float32),
        compiler_params=pltpu.CompilerParams(
            dimension_semantics=("parallel", "parallel")),
    )(sym2, par_t, pad_t, w_rev)


def kernel(ids, table, w_rev, padding):
    b, l = ids.shape
    rv = w_rev.shape[1]
    table2 = table.reshape(table.shape[0] // 2, 2 * D)
    ids_t = ids.T
    par_t = (ids_t & 1).astype(jnp.float32).reshape(l, 1, b)
    sym2 = _sc_gather(table2, ids_t)
    pad_t = padding.transpose(1, 2, 0)
    out_t = _tc_matmul(sym2, par_t, pad_t, w_rev, l, bn=4096)
    return out_t.reshape(l, rv, b).transpose(2, 0, 1)
